# baseline (speedup 1.0000x reference)
import jax
import jax.numpy as jnp
from jax import lax
from jax.experimental import pallas as pl
from jax.experimental.pallas import tpu as pltpu
from jax.experimental.pallas import tpu_sc as plsc

_N_STATES = 200000
_B = 16384
_PER = 110
_PERP = 112
_NODE_W = 72
_EDGE_W = 36
_NEG = -1000.0

_NC = 2
_NS = 16
_NW = _NC * _NS
_ROWS_PER_W = _B // _NW
_CHUNK = 256
_GSUB = 128

_NODE_CHUNKS = ((0, 1), (16, 17), (32, 33), (48, 49), (56, 57))
_EDGE_CHUNKS = ((0, 73), (16, 89), (20, 93))


def _sc_body(table, table_flat, gid, stop_m, node_m, edge_m,
             out_stop, out_node, out_edge, out_logf,
             idx_v, rows_v, node_m_v, edge_m_v, stop_m_v,
             node_o_v, edge_o_v, stop_o_v, logf_o_v, stop_raw_v,
             sidx_v, lidx_v, sem):
    wid = lax.axis_index("s") * _NC + lax.axis_index("c")

    for c in range(_ROWS_PER_W // _CHUNK):
        base = wid * _ROWS_PER_W + c * _CHUNK
        pltpu.sync_copy(gid.at[pl.ds(base, _CHUNK)], idx_v)

        def idx_body(g, carry):
            s = idx_v[pl.ds(g * 16, 16)] * _PER
            sidx_v[pl.ds(g * 16, 16)] = s
            lidx_v[pl.ds(g * 16, 16)] = s + (_PER - 1)
            return carry

        lax.fori_loop(0, _CHUNK // 16, idx_body, 0)

        cps = [
            pltpu.async_copy(table.at[idx_v.at[pl.ds(g * _GSUB, _GSUB)]],
                             rows_v.at[pl.ds(g * _GSUB, _GSUB)], sem)
            for g in range(_CHUNK // _GSUB)
        ]
        cps += [
            pltpu.async_copy(table_flat.at[sidx_v.at[pl.ds(g * _GSUB, _GSUB)]],
                             stop_raw_v.at[pl.ds(g * _GSUB, _GSUB)], sem)
            for g in range(_CHUNK // _GSUB)
        ]
        cps += [
            pltpu.async_copy(table_flat.at[lidx_v.at[pl.ds(g * _GSUB, _GSUB)]],
                             logf_o_v.at[pl.ds(g * _GSUB, _GSUB)], sem)
            for g in range(_CHUNK // _GSUB)
        ]
        pltpu.sync_copy(node_m.at[pl.ds(base, _CHUNK)], node_m_v)
        pltpu.sync_copy(edge_m.at[pl.ds(base, _CHUNK)], edge_m_v)
        pltpu.sync_copy(stop_m.at[pl.ds(base, _CHUNK)], stop_m_v)
        for cp in cps:
            cp.wait()

        def row_body(i, carry):
            for d, s in _NODE_CHUNKS:
                x = rows_v[i, pl.ds(s, 16)]
                m = node_m_v[i, pl.ds(d, 16)]
                node_o_v[i, pl.ds(d, 16)] = jnp.where(m != 0, x, _NEG)
            for d, s in _EDGE_CHUNKS:
                x = rows_v[i, pl.ds(s, 16)]
                m = edge_m_v[i, pl.ds(d, 16)]
                edge_o_v[i, pl.ds(d, 16)] = jnp.where(m != 0, x, _NEG)
            return carry

        lax.fori_loop(0, _CHUNK, row_body, 0)

        def sl_body(g, carry):
            x = stop_raw_v[pl.ds(g * 16, 16)]
            m = stop_m_v[pl.ds(g * 16, 16)]
            stop_o_v[pl.ds(g * 16, 16)] = jnp.where(m != 0, x, _NEG)
            return carry

        lax.fori_loop(0, _CHUNK // 16, sl_body, 0)

        pltpu.sync_copy(node_o_v, out_node.at[pl.ds(base, _CHUNK)])
        pltpu.sync_copy(edge_o_v, out_edge.at[pl.ds(base, _CHUNK)])
        pltpu.sync_copy(stop_o_v, out_stop.at[pl.ds(base, _CHUNK)])
        pltpu.sync_copy(logf_o_v, out_logf.at[pl.ds(base, _CHUNK)])


def kernel(gid, stop_mask, add_node_mask, add_edge_mask, table):
    table2d = jnp.pad(table.reshape(_N_STATES, _PER), ((0, 0), (0, _PERP - _PER)))
    stop_m = stop_mask.reshape(_B)
    node_m = add_node_mask.reshape(_B, _NODE_W)
    edge_m = add_edge_mask.reshape(_B, _EDGE_W)

    mesh = plsc.VectorSubcoreMesh(
        core_axis_name="c", subcore_axis_name="s", num_cores=_NC)
    fn = pl.kernel(
        _sc_body,
        mesh=mesh,
        compiler_params=pltpu.CompilerParams(use_tc_tiling_on_sc=False),
        out_type=[
            jax.ShapeDtypeStruct((_B,), jnp.float32),
            jax.ShapeDtypeStruct((_B, _NODE_W), jnp.float32),
            jax.ShapeDtypeStruct((_B, _EDGE_W), jnp.float32),
            jax.ShapeDtypeStruct((_B,), jnp.float32),
        ],
        scratch_types=[
            pltpu.VMEM((_CHUNK,), jnp.int32),
            pltpu.VMEM((_CHUNK, _PERP), jnp.float32),
            pltpu.VMEM((_CHUNK, _NODE_W), jnp.int32),
            pltpu.VMEM((_CHUNK, _EDGE_W), jnp.int32),
            pltpu.VMEM((_CHUNK,), jnp.int32),
            pltpu.VMEM((_CHUNK, _NODE_W), jnp.float32),
            pltpu.VMEM((_CHUNK, _EDGE_W), jnp.float32),
            pltpu.VMEM((_CHUNK,), jnp.float32),
            pltpu.VMEM((_CHUNK,), jnp.float32),
            pltpu.VMEM((_CHUNK,), jnp.float32),
            pltpu.VMEM((_CHUNK,), jnp.int32),
            pltpu.VMEM((_CHUNK,), jnp.int32),
            pltpu.SemaphoreType.DMA,
        ],
    )
    out_stop, out_node, out_edge, out_logf = fn(
        table2d, table, gid, stop_m, node_m, edge_m)
    return (out_stop.reshape(_B, 1),
            out_node.reshape(_B * 9, 8),
            out_edge.reshape(_B * 36, 1),
            out_logf.reshape(_B, 1))


# speedup vs baseline: 2.1784x; 2.1784x over previous
import jax
import jax.numpy as jnp
from jax import lax
from jax.experimental import pallas as pl
from jax.experimental import pallas as _pl
from jax.experimental.pallas import tpu as pltpu
from jax.experimental.pallas import tpu_sc as plsc

_N_STATES = 200000
_B = 16384
_PER = 110
_PACK = 4
_ROW_W = _PER * _PACK
_NODE_W = 72
_EDGE_W = 36
_NEG = -1000.0

_NC = 2
_NS = 16
_NW = _NC * _NS
_ROWS_PER_W = _B // _NW
_CHUNK = 128

_NODE_CHUNKS = ((0, 1), (16, 17), (32, 33), (48, 49), (56, 57))
_EDGE_CHUNKS = ((0, 73), (16, 89), (20, 93))


def _sc_body(table, gid, stop_m, node_m, edge_m,
             out_stop, out_node, out_edge, out_logf,
             idx_v, rows_v, node_m_v, edge_m_v, stop_m_v,
             node_o_v, edge_o_v, stop_o_v, logf_o_v,
             bidx_v, sem):
    wid = lax.axis_index("s") * _NC + lax.axis_index("c")
    iota = lax.iota(jnp.int32, 16)

    for c in range(_ROWS_PER_W // _CHUNK):
        base = wid * _ROWS_PER_W + c * _CHUNK
        pltpu.sync_copy(gid.at[pl.ds(base, _CHUNK)], idx_v)

        def idx_body(g, carry):
            v = idx_v[pl.ds(g * 16, 16)]
            bidx_v[pl.ds(g * 16, 16)] = lax.shift_right_logical(v, 2)
            return carry

        lax.fori_loop(0, _CHUNK // 16, idx_body, 0)

        cp = pltpu.async_copy(table.at[bidx_v], rows_v, sem)
        pltpu.sync_copy(node_m.at[pl.ds(base, _CHUNK)], node_m_v)
        pltpu.sync_copy(edge_m.at[pl.ds(base, _CHUNK)], edge_m_v)
        pltpu.sync_copy(stop_m.at[pl.ds(base, _CHUNK)], stop_m_v)
        cp.wait()

        def row_body(i, carry):
            ifull = jnp.full((16,), i, jnp.int32)
            gb = plsc.load_gather(idx_v, [ifull])
            off = (gb & (_PACK - 1)) * _PER
            for d, s in _NODE_CHUNKS:
                x = plsc.load_gather(rows_v, [ifull, off + (s + iota)])
                m = node_m_v[i, pl.ds(d, 16)]
                node_o_v[i, pl.ds(d, 16)] = jnp.where(m != 0, x, _NEG)
            for d, s in _EDGE_CHUNKS:
                x = plsc.load_gather(rows_v, [ifull, off + (s + iota)])
                m = edge_m_v[i, pl.ds(d, 16)]
                edge_o_v[i, pl.ds(d, 16)] = jnp.where(m != 0, x, _NEG)
            return carry

        lax.fori_loop(0, _CHUNK, row_body, 0)

        def sl_body(g, carry):
            rowids = g * 16 + iota
            gvec = idx_v[pl.ds(g * 16, 16)]
            off = (gvec & (_PACK - 1)) * _PER
            stopv = plsc.load_gather(rows_v, [rowids, off])
            logfv = plsc.load_gather(rows_v, [rowids, off + (_PER - 1)])
            m = stop_m_v[pl.ds(g * 16, 16)]
            stop_o_v[pl.ds(g * 16, 16)] = jnp.where(m != 0, stopv, _NEG)
            logf_o_v[pl.ds(g * 16, 16)] = logfv
            return carry

        lax.fori_loop(0, _CHUNK // 16, sl_body, 0)

        pltpu.sync_copy(node_o_v, out_node.at[pl.ds(base, _CHUNK)])
        pltpu.sync_copy(edge_o_v, out_edge.at[pl.ds(base, _CHUNK)])
        pltpu.sync_copy(stop_o_v, out_stop.at[pl.ds(base, _CHUNK)])
        pltpu.sync_copy(logf_o_v, out_logf.at[pl.ds(base, _CHUNK)])


def kernel(gid, stop_mask, add_node_mask, add_edge_mask, table):
    table440 = table.reshape(_N_STATES * _PER // _ROW_W, _ROW_W)
    stop_m = stop_mask.reshape(_B)
    node_m = add_node_mask.reshape(_B, _NODE_W)
    edge_m = add_edge_mask.reshape(_B, _EDGE_W)

    mesh = plsc.VectorSubcoreMesh(
        core_axis_name="c", subcore_axis_name="s", num_cores=_NC)
    fn = pl.kernel(
        _sc_body,
        mesh=mesh,
        compiler_params=pltpu.CompilerParams(
            use_tc_tiling_on_sc=False, needs_layout_passes=False),
        out_type=[
            jax.ShapeDtypeStruct((_B,), jnp.float32),
            jax.ShapeDtypeStruct((_B, _NODE_W), jnp.float32),
            jax.ShapeDtypeStruct((_B, _EDGE_W), jnp.float32),
            jax.ShapeDtypeStruct((_B,), jnp.float32),
        ],
        scratch_types=[
            pltpu.VMEM((_CHUNK,), jnp.int32),
            pltpu.VMEM((_CHUNK, _ROW_W), jnp.float32),
            pltpu.VMEM((_CHUNK, _NODE_W), jnp.int32),
            pltpu.VMEM((_CHUNK, _EDGE_W), jnp.int32),
            pltpu.VMEM((_CHUNK,), jnp.int32),
            pltpu.VMEM((_CHUNK, _NODE_W), jnp.float32),
            pltpu.VMEM((_CHUNK, _EDGE_W), jnp.float32),
            pltpu.VMEM((_CHUNK,), jnp.float32),
            pltpu.VMEM((_CHUNK,), jnp.float32),
            pltpu.VMEM((_CHUNK,), jnp.int32),
            pltpu.SemaphoreType.DMA,
        ],
    )
    out_stop, out_node, out_edge, out_logf = fn(
        table440, gid, stop_m, node_m, edge_m)
    return (out_stop.reshape(_B, 1),
            out_node.reshape(_B * 9, 8),
            out_edge.reshape(_B * 36, 1),
            out_logf.reshape(_B, 1))


# speedup vs baseline: 5.9783x; 2.7443x over previous
import jax
import jax.numpy as jnp
from jax import lax
from jax.experimental import pallas as pl
from jax.experimental.pallas import tpu as pltpu
from jax.experimental.pallas import tpu_sc as plsc

_N_STATES = 200000
_B = 16384
_PER = 110
_PACK = 4
_ROW_W = _PER * _PACK
_NEG = -1000.0

_N_NODE = 9
_N_TYPE = 8
_N_EDGE = 36
_NN = _B * _N_NODE
_NE = _B * _N_EDGE
_NTILE = _NN // 128

_NC = 2
_NS = 16
_NW = _NC * _NS
_ROWS_PER_W = _B // _NW
_CHUNK = 128
_NP = _CHUNK * _N_NODE * _N_TYPE
_EP = _CHUNK * _N_EDGE


def _sc_body(table, gid, stop_m, node_m, edge_m, ndiv, ediv,
             out_stop, out_node, out_edge, out_logf,
             idx_v, rows_v, node_m_v, edge_m_v, stop_m_v,
             node_o_v, edge_o_v, stop_o_v, logf_o_v,
             bidx_v, off_v, ndiv_v, ediv_v, sem):
    wid = lax.axis_index("s") * _NC + lax.axis_index("c")
    iota = lax.iota(jnp.int32, 16)

    for c in range(_ROWS_PER_W // _CHUNK):
        base = wid * _ROWS_PER_W + c * _CHUNK
        nbase = base * _N_NODE
        pbase = base * (_N_NODE * _N_TYPE)
        ebase = base * _N_EDGE
        pltpu.sync_copy(gid.at[pl.ds(base, _CHUNK)], idx_v)

        def idx_body(g, carry):
            v = idx_v[pl.ds(g * 16, 16)]
            bidx_v[pl.ds(g * 16, 16)] = lax.shift_right_logical(v, 2)
            off_v[pl.ds(g * 16, 16)] = (v & (_PACK - 1)) * _PER
            return carry

        lax.fori_loop(0, _CHUNK // 16, idx_body, 0)

        cp = pltpu.async_copy(table.at[bidx_v], rows_v, sem)
        pltpu.sync_copy(node_m.at[pl.ds(pbase, _NP)], node_m_v)
        pltpu.sync_copy(edge_m.at[pl.ds(ebase, _EP)], edge_m_v)
        pltpu.sync_copy(stop_m.at[pl.ds(base, _CHUNK)], stop_m_v)
        pltpu.sync_copy(ndiv.at[pl.ds(nbase, _CHUNK * _N_NODE)], ndiv_v)
        pltpu.sync_copy(ediv.at[pl.ds(ebase, _EP)], ediv_v)
        cp.wait()

        def node_body(q, carry):
            tt = lax.shift_right_logical(q, 6)
            ty = lax.shift_right_logical(q, 3) & 7
            j0 = (q & 7) * 16
            nloc = tt * 128 + j0
            ig = ndiv_v[pl.ds(nloc, 16)]
            il = ig - base
            k = (nbase + nloc + iota) - ig * _N_NODE
            off = plsc.load_gather(off_v, [il])
            col = off + (k * _N_TYPE + (ty + 1))
            x = plsc.load_gather(rows_v, [il, col])
            m = node_m_v[pl.ds(q * 16, 16)]
            node_o_v[pl.ds(q * 16, 16)] = jnp.where(m != 0, x, _NEG)
            return carry

        lax.fori_loop(0, _NP // 16, node_body, 0)

        def edge_body(q, carry):
            ig = ediv_v[pl.ds(q * 16, 16)]
            il = ig - base
            e = (ebase + q * 16 + iota) - ig * _N_EDGE
            off = plsc.load_gather(off_v, [il])
            col = off + (e + (1 + _N_NODE * _N_TYPE))
            x = plsc.load_gather(rows_v, [il, col])
            m = edge_m_v[pl.ds(q * 16, 16)]
            edge_o_v[pl.ds(q * 16, 16)] = jnp.where(m != 0, x, _NEG)
            return carry

        lax.fori_loop(0, _EP // 16, edge_body, 0)

        def sl_body(g, carry):
            rowids = g * 16 + iota
            off = off_v[pl.ds(g * 16, 16)]
            stopv = plsc.load_gather(rows_v, [rowids, off])
            logfv = plsc.load_gather(rows_v, [rowids, off + (_PER - 1)])
            m = stop_m_v[pl.ds(g * 16, 16)]
            stop_o_v[pl.ds(g * 16, 16)] = jnp.where(m != 0, stopv, _NEG)
            logf_o_v[pl.ds(g * 16, 16)] = logfv
            return carry

        lax.fori_loop(0, _CHUNK // 16, sl_body, 0)

        pltpu.sync_copy(node_o_v, out_node.at[pl.ds(pbase, _NP)])
        pltpu.sync_copy(edge_o_v, out_edge.at[pl.ds(ebase, _EP)])
        pltpu.sync_copy(stop_o_v, out_stop.at[pl.ds(base, _CHUNK)])
        pltpu.sync_copy(logf_o_v, out_logf.at[pl.ds(base, _CHUNK)])


def kernel(gid, stop_mask, add_node_mask, add_edge_mask, table):
    table440 = table.reshape(_N_STATES * _PER // _ROW_W, _ROW_W)
    stop_m = stop_mask.reshape(_B)
    node_m = (add_node_mask.reshape(_NTILE, 128, _N_TYPE)
              .transpose(0, 2, 1).reshape(_NN * _N_TYPE))
    edge_m = add_edge_mask.reshape(_NE)
    ndiv = jnp.arange(_NN, dtype=jnp.int32) // _N_NODE
    ediv = jnp.arange(_NE, dtype=jnp.int32) // _N_EDGE

    mesh = plsc.VectorSubcoreMesh(
        core_axis_name="c", subcore_axis_name="s", num_cores=_NC)
    fn = pl.kernel(
        _sc_body,
        mesh=mesh,
        compiler_params=pltpu.CompilerParams(
            use_tc_tiling_on_sc=False, needs_layout_passes=False),
        out_type=[
            jax.ShapeDtypeStruct((_B,), jnp.float32),
            jax.ShapeDtypeStruct((_NN * _N_TYPE,), jnp.float32),
            jax.ShapeDtypeStruct((_NE,), jnp.float32),
            jax.ShapeDtypeStruct((_B,), jnp.float32),
        ],
        scratch_types=[
            pltpu.VMEM((_CHUNK,), jnp.int32),
            pltpu.VMEM((_CHUNK, _ROW_W), jnp.float32),
            pltpu.VMEM((_NP,), jnp.int32),
            pltpu.VMEM((_EP,), jnp.int32),
            pltpu.VMEM((_CHUNK,), jnp.int32),
            pltpu.VMEM((_NP,), jnp.float32),
            pltpu.VMEM((_EP,), jnp.float32),
            pltpu.VMEM((_CHUNK,), jnp.float32),
            pltpu.VMEM((_CHUNK,), jnp.float32),
            pltpu.VMEM((_CHUNK,), jnp.int32),
            pltpu.VMEM((_CHUNK,), jnp.int32),
            pltpu.VMEM((_CHUNK * _N_NODE,), jnp.int32),
            pltpu.VMEM((_EP,), jnp.int32),
            pltpu.SemaphoreType.DMA,
        ],
    )
    out_stop, out_node, out_edge, out_logf = fn(
        table440, gid, stop_m, node_m, edge_m, ndiv, ediv)
    node_out = (out_node.reshape(_NTILE, _N_TYPE, 128)
                .transpose(0, 2, 1).reshape(_NN, _N_TYPE))
    return (out_stop.reshape(_B, 1),
            node_out,
            out_edge.reshape(_NE, 1),
            out_logf.reshape(_B, 1))


# speedup vs baseline: 6.6899x; 1.1190x over previous
import jax
import jax.numpy as jnp
from jax import lax
from jax.experimental import pallas as pl
from jax.experimental.pallas import tpu as pltpu
from jax.experimental.pallas import tpu_sc as plsc

_N_STATES = 200000
_B = 16384
_PER = 110
_BLK = 16
_BPR = 8
_NEG = -1000.0

_N_NODE = 9
_N_TYPE = 8
_N_EDGE = 36
_NN = _B * _N_NODE
_NE = _B * _N_EDGE
_NTILE = _NN // 128

_NC = 2
_NS = 16
_NW = _NC * _NS
_RPW = _B // _NW
_CHUNK = 128
_NCHUNK = _RPW // _CHUNK
_NP = _CHUNK * _N_NODE * _N_TYPE
_EP = _CHUNK * _N_EDGE
_ND = _CHUNK * _N_NODE
_NIDX = _CHUNK * _BPR


def _sc_body(table, gid, stop_m, node_m, edge_m, ndiv, ediv,
             out_stop, out_node, out_edge, out_logf,
             idx_v, b8_v, off_v,
             bidx_b, rows_b, nodem_b, nodeo_b, edgem_b, edgeo_b,
             stopm_b, stopo_b, logfo_b, ndiv_b, ediv_b,
             gsem_b, asem_b, osem_b):
    wid = lax.axis_index("s") * _NC + lax.axis_index("c")
    wbase = wid * _RPW
    iota = lax.iota(jnp.int32, 16)
    iota8 = iota * _BPR

    pltpu.sync_copy(gid.at[pl.ds(wbase, _RPW)], idx_v)

    def idx_body(g, carry):
        w = idx_v[pl.ds(g * 16, 16)] * _PER
        b8_v[pl.ds(g * 16, 16)] = lax.shift_right_logical(w, 4)
        off_v[pl.ds(g * 16, 16)] = w & (_BLK - 1)
        return carry

    lax.fori_loop(0, _RPW // 16, idx_body, 0, unroll=4)

    def issue(c, b):
        base = wbase + c * _CHUNK
        bidx_v = bidx_b[b]
        for g in range(_CHUNK // 16):
            bv = b8_v[pl.ds(c * _CHUNK + g * 16, 16)]
            pos = g * (16 * _BPR) + iota8
            for j in range(_BPR):
                plsc.store_scatter(bidx_v, [pos + j], bv + j)
        cps = [
            pltpu.async_copy(table.at[bidx_v.at[pl.ds(s * 128, 128)]],
                             rows_b[b].at[pl.ds(s * 128, 128)], gsem_b[b])
            for s in range(_NIDX // 128)
        ]
        cps.append(pltpu.async_copy(
            node_m.at[pl.ds(base * (_N_NODE * _N_TYPE), _NP)],
            nodem_b[b], asem_b[b]))
        cps.append(pltpu.async_copy(
            edge_m.at[pl.ds(base * _N_EDGE, _EP)], edgem_b[b], asem_b[b]))
        cps.append(pltpu.async_copy(
            stop_m.at[pl.ds(base, _CHUNK)], stopm_b[b], asem_b[b]))
        cps.append(pltpu.async_copy(
            ndiv.at[pl.ds(base * _N_NODE, _ND)], ndiv_b[b], asem_b[b]))
        cps.append(pltpu.async_copy(
            ediv.at[pl.ds(base * _N_EDGE, _EP)], ediv_b[b], asem_b[b]))
        return cps

    def compute(c, b):
        base = wbase + c * _CHUNK
        nbase = base * _N_NODE
        ebase = base * _N_EDGE
        rows_v = rows_b[b]
        ndiv_v, ediv_v = ndiv_b[b], ediv_b[b]
        nodem_v, nodeo_v = nodem_b[b], nodeo_b[b]
        edgem_v, edgeo_v = edgem_b[b], edgeo_b[b]

        def node_body(q, carry):
            tt = lax.shift_right_logical(q, 6)
            ty = lax.shift_right_logical(q, 3) & 7
            j0 = (q & 7) * 16
            nloc = tt * 128 + j0
            ig = ndiv_v[pl.ds(nloc, 16)]
            iw = ig - wbase
            il = iw - c * _CHUNK
            k = (nbase + nloc + iota) - ig * _N_NODE
            off = plsc.load_gather(off_v, [iw])
            w = off + (k * _N_TYPE + (ty + 1))
            row = il * _BPR + lax.shift_right_logical(w, 4)
            x = plsc.load_gather(rows_v, [row, w & (_BLK - 1)])
            m = nodem_v[pl.ds(q * 16, 16)]
            nodeo_v[pl.ds(q * 16, 16)] = jnp.where(m != 0, x, _NEG)
            return carry

        lax.fori_loop(0, _NP // 16, node_body, 0, unroll=4)

        def edge_body(q, carry):
            ig = ediv_v[pl.ds(q * 16, 16)]
            iw = ig - wbase
            il = iw - c * _CHUNK
            e = (ebase + q * 16 + iota) - ig * _N_EDGE
            off = plsc.load_gather(off_v, [iw])
            w = off + (e + (1 + _N_NODE * _N_TYPE))
            row = il * _BPR + lax.shift_right_logical(w, 4)
            x = plsc.load_gather(rows_v, [row, w & (_BLK - 1)])
            m = edgem_v[pl.ds(q * 16, 16)]
            edgeo_v[pl.ds(q * 16, 16)] = jnp.where(m != 0, x, _NEG)
            return carry

        lax.fori_loop(0, _EP // 16, edge_body, 0, unroll=4)

        def sl_body(g, carry):
            il = g * 16 + iota
            off = off_v[pl.ds(c * _CHUNK + g * 16, 16)]
            il8 = il * _BPR
            stopv = plsc.load_gather(rows_v, [il8, off])
            wl = off + (_PER - 1)
            rowl = il8 + lax.shift_right_logical(wl, 4)
            logfv = plsc.load_gather(rows_v, [rowl, wl & (_BLK - 1)])
            m = stopm_b[b][pl.ds(g * 16, 16)]
            stopo_b[b][pl.ds(g * 16, 16)] = jnp.where(m != 0, stopv, _NEG)
            logfo_b[b][pl.ds(g * 16, 16)] = logfv
            return carry

        lax.fori_loop(0, _CHUNK // 16, sl_body, 0, unroll=4)

        return [
            pltpu.async_copy(nodeo_v, out_node.at[pl.ds(nbase * _N_TYPE, _NP)],
                             osem_b[b]),
            pltpu.async_copy(edgeo_v, out_edge.at[pl.ds(ebase, _EP)],
                             osem_b[b]),
            pltpu.async_copy(stopo_b[b], out_stop.at[pl.ds(base, _CHUNK)],
                             osem_b[b]),
            pltpu.async_copy(logfo_b[b], out_logf.at[pl.ds(base, _CHUNK)],
                             osem_b[b]),
        ]

    pending = {0: [], 1: []}
    outs = {0: [], 1: []}
    pending[0] = issue(0, 0)
    for c in range(_NCHUNK):
        b = c % 2
        if c + 1 < _NCHUNK:
            pending[1 - b] = issue(c + 1, 1 - b)
        for cp in pending[b]:
            cp.wait()
        for cp in outs[b]:
            cp.wait()
        outs[b] = compute(c, b)
    for b in (0, 1):
        for cp in outs[b]:
            cp.wait()


def kernel(gid, stop_mask, add_node_mask, add_edge_mask, table):
    table16 = table.reshape(_N_STATES * _PER // _BLK, _BLK)
    stop_m = stop_mask.reshape(_B)
    node_m = (add_node_mask.reshape(_NTILE, 128, _N_TYPE)
              .transpose(0, 2, 1).reshape(_NN * _N_TYPE))
    edge_m = add_edge_mask.reshape(_NE)
    ndiv = jnp.arange(_NN, dtype=jnp.int32) // _N_NODE
    ediv = jnp.arange(_NE, dtype=jnp.int32) // _N_EDGE

    mesh = plsc.VectorSubcoreMesh(
        core_axis_name="c", subcore_axis_name="s", num_cores=_NC)
    fn = pl.kernel(
        _sc_body,
        mesh=mesh,
        compiler_params=pltpu.CompilerParams(
            use_tc_tiling_on_sc=False, needs_layout_passes=False),
        out_type=[
            jax.ShapeDtypeStruct((_B,), jnp.float32),
            jax.ShapeDtypeStruct((_NN * _N_TYPE,), jnp.float32),
            jax.ShapeDtypeStruct((_NE,), jnp.float32),
            jax.ShapeDtypeStruct((_B,), jnp.float32),
        ],
        scratch_types=[
            pltpu.VMEM((_RPW,), jnp.int32),
            pltpu.VMEM((_RPW,), jnp.int32),
            pltpu.VMEM((_RPW,), jnp.int32),
            [pltpu.VMEM((_NIDX,), jnp.int32) for _ in range(2)],
            [pltpu.VMEM((_NIDX, _BLK), jnp.float32) for _ in range(2)],
            [pltpu.VMEM((_NP,), jnp.int32) for _ in range(2)],
            [pltpu.VMEM((_NP,), jnp.float32) for _ in range(2)],
            [pltpu.VMEM((_EP,), jnp.int32) for _ in range(2)],
            [pltpu.VMEM((_EP,), jnp.float32) for _ in range(2)],
            [pltpu.VMEM((_CHUNK,), jnp.int32) for _ in range(2)],
            [pltpu.VMEM((_CHUNK,), jnp.float32) for _ in range(2)],
            [pltpu.VMEM((_CHUNK,), jnp.float32) for _ in range(2)],
            [pltpu.VMEM((_ND,), jnp.int32) for _ in range(2)],
            [pltpu.VMEM((_EP,), jnp.int32) for _ in range(2)],
            [pltpu.SemaphoreType.DMA for _ in range(2)],
            [pltpu.SemaphoreType.DMA for _ in range(2)],
            [pltpu.SemaphoreType.DMA for _ in range(2)],
        ],
    )
    out_stop, out_node, out_edge, out_logf = fn(
        table16, gid, stop_m, node_m, edge_m, ndiv, ediv)
    node_out = (out_node.reshape(_NTILE, _N_TYPE, 128)
                .transpose(0, 2, 1).reshape(_NN, _N_TYPE))
    return (out_stop.reshape(_B, 1),
            node_out,
            out_edge.reshape(_NE, 1),
            out_logf.reshape(_B, 1))


# speedup vs baseline: 15.5912x; 2.3306x over previous
import jax
import jax.numpy as jnp
from jax import lax
from jax.experimental import pallas as pl
from jax.experimental.pallas import tpu as pltpu
from jax.experimental.pallas import tpu_sc as plsc

_N_STATES = 200000
_B = 16384
_PER = 110
_BLK = 16
_BPR = 8
_NEG = -1000.0

_N_NODE = 9
_N_TYPE = 8
_N_EDGE = 36
_NN = _B * _N_NODE
_NE = _B * _N_EDGE
_NTILE = _NN // 128

_NC = 2
_NS = 16
_NW = _NC * _NS
_RPW = _B // _NW
_CHUNK = 128
_NCHUNK = _RPW // _CHUNK
_NP = _CHUNK * _N_NODE * _N_TYPE
_EP = _CHUNK * _N_EDGE
_ND = _CHUNK * _N_NODE
_NIDX = _CHUNK * _BPR


def _sc_body(table, gid, stop_m, node_m, edge_m, ndiv, ediv,
             out_stop, out_node, out_edge, out_logf,
             idx_v, b8_v, off_v,
             bidx_b, rows_b, nodem_b, nodeo_b, edgem_b, edgeo_b,
             stopm_b, stopo_b, logfo_b, ndiv_b, ediv_b,
             gsem_b, asem_b, osem_b):
    wid = lax.axis_index("s") * _NC + lax.axis_index("c")
    wbase = wid * _RPW
    iota = lax.iota(jnp.int32, 16)
    iota8 = iota * _BPR

    pltpu.sync_copy(gid.at[pl.ds(wbase, _RPW)], idx_v)

    @plsc.parallel_loop(0, _RPW // 16, unroll=4)
    def idx_body(g):
        w = idx_v[pl.ds(g * 16, 16)] * _PER
        b8_v[pl.ds(g * 16, 16)] = lax.shift_right_logical(w, 4)
        off_v[pl.ds(g * 16, 16)] = w & (_BLK - 1)

    def issue(c, b):
        base = wbase + c * _CHUNK
        bidx_v = bidx_b[b]
        for g in range(_CHUNK // 16):
            bv = b8_v[pl.ds(c * _CHUNK + g * 16, 16)]
            pos = g * (16 * _BPR) + iota8
            for j in range(_BPR):
                plsc.store_scatter(bidx_v, [pos + j], bv + j)
        cps = [
            pltpu.async_copy(table.at[bidx_v.at[pl.ds(s * 128, 128)]],
                             rows_b[b].at[pl.ds(s * 128, 128)], gsem_b[b])
            for s in range(_NIDX // 128)
        ]
        cps.append(pltpu.async_copy(
            node_m.at[pl.ds(base * (_N_NODE * _N_TYPE), _NP)],
            nodem_b[b], asem_b[b]))
        cps.append(pltpu.async_copy(
            edge_m.at[pl.ds(base * _N_EDGE, _EP)], edgem_b[b], asem_b[b]))
        cps.append(pltpu.async_copy(
            stop_m.at[pl.ds(base, _CHUNK)], stopm_b[b], asem_b[b]))
        cps.append(pltpu.async_copy(
            ndiv.at[pl.ds(base * _N_NODE, _ND)], ndiv_b[b], asem_b[b]))
        cps.append(pltpu.async_copy(
            ediv.at[pl.ds(base * _N_EDGE, _EP)], ediv_b[b], asem_b[b]))
        return cps

    def compute(c, b):
        base = wbase + c * _CHUNK
        nbase = base * _N_NODE
        ebase = base * _N_EDGE
        rows_v = rows_b[b]
        ndiv_v, ediv_v = ndiv_b[b], ediv_b[b]
        nodem_v, nodeo_v = nodem_b[b], nodeo_b[b]
        edgem_v, edgeo_v = edgem_b[b], edgeo_b[b]

        @plsc.parallel_loop(0, _NP // 16, unroll=8)
        def node_body(q):
            tt = lax.shift_right_logical(q, 6)
            ty = lax.shift_right_logical(q, 3) & 7
            j0 = (q & 7) * 16
            nloc = tt * 128 + j0
            ig = ndiv_v[pl.ds(nloc, 16)]
            iw = ig - wbase
            il = iw - c * _CHUNK
            k = (nbase + nloc + iota) - ig * _N_NODE
            off = plsc.load_gather(off_v, [iw])
            w = off + (k * _N_TYPE + (ty + 1))
            row = il * _BPR + lax.shift_right_logical(w, 4)
            x = plsc.load_gather(rows_v, [row, w & (_BLK - 1)])
            m = nodem_v[pl.ds(q * 16, 16)]
            nodeo_v[pl.ds(q * 16, 16)] = jnp.where(m != 0, x, _NEG)

        @plsc.parallel_loop(0, _EP // 16, unroll=8)
        def edge_body(q):
            ig = ediv_v[pl.ds(q * 16, 16)]
            iw = ig - wbase
            il = iw - c * _CHUNK
            e = (ebase + q * 16 + iota) - ig * _N_EDGE
            off = plsc.load_gather(off_v, [iw])
            w = off + (e + (1 + _N_NODE * _N_TYPE))
            row = il * _BPR + lax.shift_right_logical(w, 4)
            x = plsc.load_gather(rows_v, [row, w & (_BLK - 1)])
            m = edgem_v[pl.ds(q * 16, 16)]
            edgeo_v[pl.ds(q * 16, 16)] = jnp.where(m != 0, x, _NEG)

        @plsc.parallel_loop(0, _CHUNK // 16, unroll=4)
        def sl_body(g):
            il = g * 16 + iota
            off = off_v[pl.ds(c * _CHUNK + g * 16, 16)]
            il8 = il * _BPR
            stopv = plsc.load_gather(rows_v, [il8, off])
            wl = off + (_PER - 1)
            rowl = il8 + lax.shift_right_logical(wl, 4)
            logfv = plsc.load_gather(rows_v, [rowl, wl & (_BLK - 1)])
            m = stopm_b[b][pl.ds(g * 16, 16)]
            stopo_b[b][pl.ds(g * 16, 16)] = jnp.where(m != 0, stopv, _NEG)
            logfo_b[b][pl.ds(g * 16, 16)] = logfv

        return [
            pltpu.async_copy(nodeo_v, out_node.at[pl.ds(nbase * _N_TYPE, _NP)],
                             osem_b[b]),
            pltpu.async_copy(edgeo_v, out_edge.at[pl.ds(ebase, _EP)],
                             osem_b[b]),
            pltpu.async_copy(stopo_b[b], out_stop.at[pl.ds(base, _CHUNK)],
                             osem_b[b]),
            pltpu.async_copy(logfo_b[b], out_logf.at[pl.ds(base, _CHUNK)],
                             osem_b[b]),
        ]

    pending = {0: [], 1: []}
    outs = {0: [], 1: []}
    pending[0] = issue(0, 0)
    for c in range(_NCHUNK):
        b = c % 2
        if c + 1 < _NCHUNK:
            pending[1 - b] = issue(c + 1, 1 - b)
        for cp in pending[b]:
            cp.wait()
        for cp in outs[b]:
            cp.wait()
        outs[b] = compute(c, b)
    for b in (0, 1):
        for cp in outs[b]:
            cp.wait()


def kernel(gid, stop_mask, add_node_mask, add_edge_mask, table):
    table16 = table.reshape(_N_STATES * _PER // _BLK, _BLK)
    stop_m = stop_mask.reshape(_B)
    node_m = (add_node_mask.reshape(_NTILE, 128, _N_TYPE)
              .transpose(0, 2, 1).reshape(_NN * _N_TYPE))
    edge_m = add_edge_mask.reshape(_NE)
    ndiv = jnp.arange(_NN, dtype=jnp.int32) // _N_NODE
    ediv = jnp.arange(_NE, dtype=jnp.int32) // _N_EDGE

    mesh = plsc.VectorSubcoreMesh(
        core_axis_name="c", subcore_axis_name="s", num_cores=_NC)
    fn = pl.kernel(
        _sc_body,
        mesh=mesh,
        compiler_params=pltpu.CompilerParams(
            use_tc_tiling_on_sc=False, needs_layout_passes=False),
        out_type=[
            jax.ShapeDtypeStruct((_B,), jnp.float32),
            jax.ShapeDtypeStruct((_NN * _N_TYPE,), jnp.float32),
            jax.ShapeDtypeStruct((_NE,), jnp.float32),
            jax.ShapeDtypeStruct((_B,), jnp.float32),
        ],
        scratch_types=[
            pltpu.VMEM((_RPW,), jnp.int32),
            pltpu.VMEM((_RPW,), jnp.int32),
            pltpu.VMEM((_RPW,), jnp.int32),
            [pltpu.VMEM((_NIDX,), jnp.int32) for _ in range(2)],
            [pltpu.VMEM((_NIDX, _BLK), jnp.float32) for _ in range(2)],
            [pltpu.VMEM((_NP,), jnp.int32) for _ in range(2)],
            [pltpu.VMEM((_NP,), jnp.float32) for _ in range(2)],
            [pltpu.VMEM((_EP,), jnp.int32) for _ in range(2)],
            [pltpu.VMEM((_EP,), jnp.float32) for _ in range(2)],
            [pltpu.VMEM((_CHUNK,), jnp.int32) for _ in range(2)],
            [pltpu.VMEM((_CHUNK,), jnp.float32) for _ in range(2)],
            [pltpu.VMEM((_CHUNK,), jnp.float32) for _ in range(2)],
            [pltpu.VMEM((_ND,), jnp.int32) for _ in range(2)],
            [pltpu.VMEM((_EP,), jnp.int32) for _ in range(2)],
            [pltpu.SemaphoreType.DMA for _ in range(2)],
            [pltpu.SemaphoreType.DMA for _ in range(2)],
            [pltpu.SemaphoreType.DMA for _ in range(2)],
        ],
    )
    out_stop, out_node, out_edge, out_logf = fn(
        table16, gid, stop_m, node_m, edge_m, ndiv, ediv)
    node_out = (out_node.reshape(_NTILE, _N_TYPE, 128)
                .transpose(0, 2, 1).reshape(_NN, _N_TYPE))
    return (out_stop.reshape(_B, 1),
            node_out,
            out_edge.reshape(_NE, 1),
            out_logf.reshape(_B, 1))
